# Initial kernel scaffold; baseline (speedup 1.0000x reference)
import functools

import jax
import jax.numpy as jnp
from jax import lax
from jax.experimental import pallas as pl
from jax.experimental.pallas import tpu as pltpu
from jax.experimental.pallas import tpu_sc as plsc

B = 16384
V = 100000
F_CAT = 26
F_NUM = 13
D = 16

NC = 2
NS = 16
NW = NC * NS
TOTAL_ROWS = F_CAT * B
R_PER_W = TOTAL_ROWS // NW
CHUNK = 1664
NCHUNK = R_PER_W // CHUNK


def _sc_body(gidx_hbm, emb_tab_hbm, lin_tab_hbm, emb_out_hbm, lin_out_hbm,
             idx_v, rows_v, lin_v, sem_e, sem_l):
    wid = lax.axis_index("s") * NC + lax.axis_index("c")
    base = wid * R_PER_W
    for c in range(NCHUNK):
        off = base + c * CHUNK
        pltpu.sync_copy(gidx_hbm.at[pl.ds(off, CHUNK)], idx_v)
        ce = pltpu.async_copy(emb_tab_hbm.at[idx_v], rows_v, sem_e)
        cl = pltpu.async_copy(lin_tab_hbm.at[idx_v], lin_v, sem_l)
        ce.wait()
        cl.wait()
        pltpu.sync_copy(rows_v, emb_out_hbm.at[pl.ds(off, CHUNK)])
        pltpu.sync_copy(lin_v, lin_out_hbm.at[pl.ds(off, CHUNK)])


_sc_gather = functools.partial(
    pl.kernel,
    out_type=[
        jax.ShapeDtypeStruct((TOTAL_ROWS, D), jnp.float32),
        jax.ShapeDtypeStruct((TOTAL_ROWS, 1), jnp.float32),
    ],
    mesh=plsc.VectorSubcoreMesh(core_axis_name="c", subcore_axis_name="s"),
    scratch_types=[
        pltpu.VMEM((CHUNK,), jnp.int32),
        pltpu.VMEM((CHUNK, D), jnp.float32),
        pltpu.VMEM((CHUNK, 1), jnp.float32),
        pltpu.SemaphoreType.DMA,
        pltpu.SemaphoreType.DMA,
    ],
)(_sc_body)


BT = 2048


def _mlp_body(emb_ref, lin_ref, num_ref, cw_ref, w1_ref, b1_ref, w2_ref,
              b2_ref, w3_ref, b3_ref, w4_ref, b4_ref, out_ref):
    acc = jnp.zeros((BT, 64), jnp.float32)
    for f in range(F_CAT):
        acc = acc + jnp.dot(emb_ref[f], w1_ref[f * D:(f + 1) * D, :],
                            preferred_element_type=jnp.float32)
    num_blk = num_ref[...]
    w1n = w1_ref[F_CAT * D:F_CAT * D + F_NUM, :]
    acc = acc + lax.dot_general(num_blk, w1n, (((0,), (0,)), ((), ())),
                                preferred_element_type=jnp.float32)
    h = jnp.maximum(acc + b1_ref[...], 0.0)
    h = jnp.maximum(jnp.dot(h, w2_ref[...], preferred_element_type=jnp.float32)
                    + b2_ref[...], 0.0)
    h = jnp.maximum(jnp.dot(h, w3_ref[...], preferred_element_type=jnp.float32)
                    + b3_ref[...], 0.0)
    deep = jnp.dot(h, w4_ref[...], preferred_element_type=jnp.float32) + b4_ref[...]
    lin_sum = jnp.sum(lin_ref[...], axis=0) + jnp.sum(cw_ref[...] * num_blk, axis=0)
    z = deep + lin_sum[:, None]
    out_ref[...] = 1.0 / (1.0 + jnp.exp(-z))


def _mlp_call(emb_rows, lin_rows, num, cw, W1, b1, W2, b2, W3, b3, W4, b4):
    grid = (B // BT,)

    def full(shape):
        return pl.BlockSpec(shape, lambda *_: tuple(0 for _ in shape))

    return pl.pallas_call(
        _mlp_body,
        grid=grid,
        in_specs=[
            pl.BlockSpec((F_CAT, BT, D), lambda i: (0, i, 0)),
            pl.BlockSpec((F_CAT, BT), lambda i: (0, i)),
            pl.BlockSpec((F_NUM, BT), lambda i: (0, i)),
            full((F_NUM, 1)),
            full((F_CAT * D + F_NUM, 64)),
            full((1, 64)),
            full((64, 32)),
            full((1, 32)),
            full((32, 16)),
            full((1, 16)),
            full((16, 1)),
            full((1, 1)),
        ],
        out_specs=pl.BlockSpec((BT, 1), lambda i: (i, 0)),
        out_shape=jax.ShapeDtypeStruct((B, 1), jnp.float32),
    )(emb_rows, lin_rows, num, cw, W1, b1, W2, b2, W3, b3, W4, b4)


def kernel(cat, num, lin_tables, emb_tables, cont_w, W1, b1, W2, b2, W3, b3,
           W4, b4):
    offs = (jnp.arange(F_CAT, dtype=jnp.int32) * V)[:, None]
    gidx = (cat.astype(jnp.int32) + offs).reshape(-1)
    emb_flat = emb_tables.reshape(F_CAT * V, D)
    lin_flat = lin_tables.reshape(F_CAT * V, 1)
    emb_rows, lin_rows = _sc_gather(gidx, emb_flat, lin_flat)
    return _mlp_call(
        emb_rows.reshape(F_CAT, B, D),
        lin_rows.reshape(F_CAT, B),
        num,
        cont_w.reshape(F_NUM, 1),
        W1,
        b1.reshape(1, -1),
        W2,
        b2.reshape(1, -1),
        W3,
        b3.reshape(1, -1),
        W4,
        b4.reshape(1, -1),
    )


# speedup vs baseline: 6.8202x; 6.8202x over previous
import functools

import jax
import jax.numpy as jnp
from jax import lax
from jax.experimental import pallas as pl
from jax.experimental.pallas import tpu as pltpu
from jax.experimental.pallas import tpu_sc as plsc

B = 16384
V = 100000
F_CAT = 26
F_NUM = 13
D = 16

NC = 2
NS = 16
NW = NC * NS
TOTAL_ROWS = F_CAT * B
R_PER_W = TOTAL_ROWS // NW
SUB = 128
NSUB = 13
CHUNK = SUB * NSUB
NCHUNK = R_PER_W // CHUNK
IDX_ROWS_PER_W = R_PER_W // SUB


def _sc_body(gidx_hbm, emb_tab_hbm, lin_tab_hbm, emb_out_hbm, lin_out_hbm,
             idx_v, rows_v, lin_v, sem_e, sem_l):
    wid = lax.axis_index("s") * NC + lax.axis_index("c")
    base = wid * R_PER_W
    idx_base = wid * IDX_ROWS_PER_W
    for c in range(NCHUNK):
        off = base + c * CHUNK
        pltpu.sync_copy(gidx_hbm.at[pl.ds(idx_base + c * NSUB, NSUB)], idx_v)
        copies = []
        for j in range(NSUB):
            copies.append(pltpu.async_copy(
                emb_tab_hbm.at[idx_v.at[j]],
                rows_v.at[pl.ds(j * SUB, SUB)], sem_e))
            copies.append(pltpu.async_copy(
                lin_tab_hbm.at[idx_v.at[j]],
                lin_v.at[pl.ds(j * SUB, SUB)], sem_l))
        for cp in copies:
            cp.wait()
        pltpu.sync_copy(rows_v, emb_out_hbm.at[pl.ds(off, CHUNK)])
        pltpu.sync_copy(lin_v, lin_out_hbm.at[pl.ds(off, CHUNK)])


_sc_gather = functools.partial(
    pl.kernel,
    out_type=[
        jax.ShapeDtypeStruct((TOTAL_ROWS, D), jnp.float32),
        jax.ShapeDtypeStruct((TOTAL_ROWS,), jnp.float32),
    ],
    mesh=plsc.VectorSubcoreMesh(core_axis_name="c", subcore_axis_name="s"),
    scratch_types=[
        pltpu.VMEM((NSUB, SUB), jnp.int32),
        pltpu.VMEM((CHUNK, D), jnp.float32),
        pltpu.VMEM((CHUNK,), jnp.float32),
        pltpu.SemaphoreType.DMA,
        pltpu.SemaphoreType.DMA,
    ],
    compiler_params=pltpu.CompilerParams(use_tc_tiling_on_sc=False),
)(_sc_body)


BT = 2048


def _mlp_body(emb_ref, lin_ref, num_ref, cw_ref, w1_ref, b1_ref, w2_ref,
              b2_ref, w3_ref, b3_ref, w4_ref, b4_ref, out_ref):
    acc = jnp.zeros((BT, 64), jnp.float32)
    for f in range(F_CAT):
        acc = acc + jnp.dot(emb_ref[f], w1_ref[f * D:(f + 1) * D, :],
                            preferred_element_type=jnp.float32)
    num_blk = num_ref[...]
    w1n = w1_ref[F_CAT * D:F_CAT * D + F_NUM, :]
    acc = acc + lax.dot_general(num_blk, w1n, (((0,), (0,)), ((), ())),
                                preferred_element_type=jnp.float32)
    h = jnp.maximum(acc + b1_ref[...], 0.0)
    h = jnp.maximum(jnp.dot(h, w2_ref[...], preferred_element_type=jnp.float32)
                    + b2_ref[...], 0.0)
    h = jnp.maximum(jnp.dot(h, w3_ref[...], preferred_element_type=jnp.float32)
                    + b3_ref[...], 0.0)
    deep = jnp.dot(h, w4_ref[...], preferred_element_type=jnp.float32) + b4_ref[...]
    lin_sum = jnp.sum(lin_ref[...], axis=0) + jnp.sum(cw_ref[...] * num_blk, axis=0)
    z = deep + lin_sum[:, None]
    out_ref[...] = 1.0 / (1.0 + jnp.exp(-z))


def _mlp_call(emb_rows, lin_rows, num, cw, W1, b1, W2, b2, W3, b3, W4, b4):
    grid = (B // BT,)

    def full(shape):
        return pl.BlockSpec(shape, lambda *_: tuple(0 for _ in shape))

    return pl.pallas_call(
        _mlp_body,
        grid=grid,
        in_specs=[
            pl.BlockSpec((F_CAT, BT, D), lambda i: (0, i, 0)),
            pl.BlockSpec((F_CAT, BT), lambda i: (0, i)),
            pl.BlockSpec((F_NUM, BT), lambda i: (0, i)),
            full((F_NUM, 1)),
            full((F_CAT * D + F_NUM, 64)),
            full((1, 64)),
            full((64, 32)),
            full((1, 32)),
            full((32, 16)),
            full((1, 16)),
            full((16, 1)),
            full((1, 1)),
        ],
        out_specs=pl.BlockSpec((BT, 1), lambda i: (i, 0)),
        out_shape=jax.ShapeDtypeStruct((B, 1), jnp.float32),
    )(emb_rows, lin_rows, num, cw, W1, b1, W2, b2, W3, b3, W4, b4)


def kernel(cat, num, lin_tables, emb_tables, cont_w, W1, b1, W2, b2, W3, b3,
           W4, b4):
    offs = (jnp.arange(F_CAT, dtype=jnp.int32) * V)[:, None]
    gidx = (cat.astype(jnp.int32) + offs).reshape(TOTAL_ROWS // SUB, SUB)
    emb_flat = emb_tables.reshape(F_CAT * V, D)
    lin_flat = lin_tables.reshape(F_CAT * V)
    emb_rows, lin_rows = _sc_gather(gidx, emb_flat, lin_flat)
    return _mlp_call(
        emb_rows.reshape(F_CAT, B, D),
        lin_rows.reshape(F_CAT, B),
        num,
        cont_w.reshape(F_NUM, 1),
        W1,
        b1.reshape(1, -1),
        W2,
        b2.reshape(1, -1),
        W3,
        b3.reshape(1, -1),
        W4,
        b4.reshape(1, -1),
    )


# speedup vs baseline: 9.2635x; 1.3582x over previous
import functools

import jax
import jax.numpy as jnp
from jax import lax
from jax.experimental import pallas as pl
from jax.experimental.pallas import tpu as pltpu
from jax.experimental.pallas import tpu_sc as plsc

B = 16384
V = 100000
F_CAT = 26
F_NUM = 13
D = 16

PACK_VB = 33408
PACK_PR = PACK_VB // 8
PACK_NVB = 3
FSTRIDE = PACK_VB * PACK_NVB
PACK_ROWS = FSTRIDE // 8
EMB_ROWS_PAD = F_CAT * FSTRIDE


def _pack_body(x_ref, o_ref):
    xt = jnp.transpose(x_ref[...])
    xt3 = xt.reshape(PACK_PR, 8, 16)
    o_ref[0] = jnp.concatenate([xt3[:, j, :] for j in range(8)], axis=-1)


_pack_call = pl.pallas_call(
    _pack_body,
    grid=(F_CAT, PACK_NVB),
    in_specs=[pl.BlockSpec((D, PACK_VB), lambda f, v: (f, v))],
    out_specs=pl.BlockSpec((1, PACK_PR, 128), lambda f, v: (f, v, 0)),
    out_shape=jax.ShapeDtypeStruct((F_CAT, PACK_NVB * PACK_PR, 128),
                                   jnp.float32),
)

NC = 2
NS = 16
NW = NC * NS
TOTAL_ROWS = F_CAT * B
R_PER_W = TOTAL_ROWS // NW
SUB = 128
NSUB = 13
CHUNK = SUB * NSUB
NCHUNK = R_PER_W // CHUNK
IDX_ROWS_PER_W = R_PER_W // SUB


def _sc_body(gide_hbm, gidl_hbm, emb_tab_hbm, lin_tab_hbm,
             emb_out_hbm, lin_out_hbm,
             idxe_v, idxl_v, rows_v, lin_v, sem_e, sem_l):
    wid = lax.axis_index("s") * NC + lax.axis_index("c")
    base = wid * R_PER_W
    idx_base = wid * IDX_ROWS_PER_W
    for c in range(NCHUNK):
        off = base + c * CHUNK
        pltpu.sync_copy(gide_hbm.at[pl.ds(idx_base + c * NSUB, NSUB)], idxe_v)
        pltpu.sync_copy(gidl_hbm.at[pl.ds(idx_base + c * NSUB, NSUB)], idxl_v)
        copies = []
        for j in range(NSUB):
            copies.append(pltpu.async_copy(
                emb_tab_hbm.at[idxe_v.at[j]],
                rows_v.at[pl.ds(j * SUB, SUB)], sem_e))
            copies.append(pltpu.async_copy(
                lin_tab_hbm.at[idxl_v.at[j]],
                lin_v.at[pl.ds(j * SUB, SUB)], sem_l))
        for cp in copies:
            cp.wait()
        pltpu.sync_copy(rows_v, emb_out_hbm.at[pl.ds(off, CHUNK)])
        pltpu.sync_copy(lin_v, lin_out_hbm.at[pl.ds(off, CHUNK)])


_sc_gather = functools.partial(
    pl.kernel,
    out_type=[
        jax.ShapeDtypeStruct((TOTAL_ROWS, D), jnp.float32),
        jax.ShapeDtypeStruct((TOTAL_ROWS,), jnp.float32),
    ],
    mesh=plsc.VectorSubcoreMesh(core_axis_name="c", subcore_axis_name="s"),
    scratch_types=[
        pltpu.VMEM((NSUB, SUB), jnp.int32),
        pltpu.VMEM((NSUB, SUB), jnp.int32),
        pltpu.VMEM((CHUNK, D), jnp.float32),
        pltpu.VMEM((CHUNK,), jnp.float32),
        pltpu.SemaphoreType.DMA,
        pltpu.SemaphoreType.DMA,
    ],
    compiler_params=pltpu.CompilerParams(use_tc_tiling_on_sc=False),
)(_sc_body)

BT = 2048
PRT = BT // 8


def _mlp_body(embp_ref, linp_ref, nump_ref, w1e_ref, w1n_ref, w2_ref, w3_ref,
              w4_ref, cwb_ref, b1_ref, b2_ref, b3_ref, b4_ref, out_ref):
    f32 = jnp.float32
    npk = nump_ref[...]
    acc = jnp.dot(npk, w1n_ref[...], preferred_element_type=f32)
    for f in range(F_CAT):
        acc = acc + jnp.dot(embp_ref[f], w1e_ref[f],
                            preferred_element_type=f32)
    h = jnp.maximum(acc + b1_ref[...], 0.0)
    h = jnp.maximum(jnp.dot(h, w2_ref[...], preferred_element_type=f32)
                    + b2_ref[...], 0.0)
    h = jnp.maximum(jnp.dot(h, w3_ref[...], preferred_element_type=f32)
                    + b3_ref[...], 0.0)
    z = jnp.dot(h, w4_ref[...], preferred_element_type=f32) + b4_ref[...]
    lin_s = linp_ref[0]
    for f in range(1, F_CAT):
        lin_s = lin_s + linp_ref[f]
    z = z + lin_s + jnp.dot(npk, cwb_ref[...], preferred_element_type=f32)
    out_ref[...] = 1.0 / (1.0 + jnp.exp(-z))


def _mlp_call(embp, linp, nump, w1e, w1n, w2, w3, w4, cwb, b1, b2, b3, b4):
    def full(shape):
        return pl.BlockSpec(shape, lambda *_: tuple(0 for _ in shape))

    return pl.pallas_call(
        _mlp_body,
        grid=(B // BT,),
        in_specs=[
            pl.BlockSpec((F_CAT, PRT, 128), lambda i: (0, i, 0)),
            pl.BlockSpec((F_CAT, PRT, 8), lambda i: (0, i, 0)),
            pl.BlockSpec((PRT, 128), lambda i: (i, 0)),
            full((F_CAT, 128, 512)),
            full((128, 512)),
            full((512, 256)),
            full((256, 128)),
            full((128, 8)),
            full((128, 8)),
            full((1, 512)),
            full((1, 256)),
            full((1, 128)),
            full((1, 8)),
        ],
        out_specs=pl.BlockSpec((PRT, 8), lambda i: (i, 0)),
        out_shape=jax.ShapeDtypeStruct((B // 8, 8), jnp.float32),
    )(embp, linp, nump, w1e, w1n, w2, w3, w4, cwb, b1, b2, b3, b4)


def kernel(cat, num, lin_tables, emb_tables, cont_w, W1, b1, W2, b2, W3, b3,
           W4, b4):
    emb_fm = jnp.transpose(emb_tables, (0, 2, 1)).reshape(F_CAT * D, V)
    packed = _pack_call(emb_fm)
    emb_tab = packed.reshape(EMB_ROWS_PAD, D)
    lin_flat = lin_tables.reshape(F_CAT * V)
    cat32 = cat.astype(jnp.int32)
    gidx_e = (cat32 + (jnp.arange(F_CAT, dtype=jnp.int32) * FSTRIDE)[:, None]
              ).reshape(TOTAL_ROWS // SUB, SUB)
    gidx_l = (cat32 + (jnp.arange(F_CAT, dtype=jnp.int32) * V)[:, None]
              ).reshape(TOTAL_ROWS // SUB, SUB)
    emb_rows, lin_rows = _sc_gather(gidx_e, gidx_l, emb_tab, lin_flat)
    f32 = jnp.float32
    eye8 = jnp.eye(8, dtype=f32)
    W1e3 = W1[:F_CAT * D].reshape(F_CAT, D, 64)
    Wb1e = jnp.einsum('jk,fdo->fjdko', eye8, W1e3).reshape(F_CAT, 128, 512)
    W1n = jnp.pad(W1[F_CAT * D:], ((0, 3), (0, 0)))
    Wb1n = jnp.einsum('jk,do->jdko', eye8, W1n).reshape(128, 512)
    Wb2 = jnp.einsum('jk,do->jdko', eye8, W2).reshape(512, 256)
    Wb3 = jnp.einsum('jk,do->jdko', eye8, W3).reshape(256, 128)
    Wb4 = jnp.einsum('jk,do->jdko', eye8, W4).reshape(128, 8)
    cwB = jnp.einsum('jk,c->jck', eye8, jnp.pad(cont_w, (0, 3))
                     ).reshape(128, 8)
    b1t = jnp.tile(b1, 8)[None]
    b2t = jnp.tile(b2, 8)[None]
    b3t = jnp.tile(b3, 8)[None]
    b4t = jnp.tile(b4, 8)[None]
    numP = jnp.pad(num.T, ((0, 0), (0, 3))).reshape(B // 8, 128)
    out8 = _mlp_call(
        emb_rows.reshape(F_CAT, B // 8, 128),
        lin_rows.reshape(F_CAT, B // 8, 8),
        numP, Wb1e, Wb1n, Wb2, Wb3, Wb4, cwB, b1t, b2t, b3t, b4t)
    return out8.reshape(B, 1)


# speedup vs baseline: 31.2460x; 3.3730x over previous
import functools

import jax
import jax.numpy as jnp
from jax import lax
from jax.experimental import pallas as pl
from jax.experimental.pallas import tpu as pltpu
from jax.experimental.pallas import tpu_sc as plsc

B = 16384
V = 100000
F_CAT = 26
F_NUM = 13
D = 16

PACK_VB = 33792
PACK_PR = PACK_VB // 8
PACK_NVB = 3
FSTRIDE = PACK_VB * PACK_NVB
PACK_ROWS = FSTRIDE // 8
EMB_ROWS_PAD = F_CAT * FSTRIDE


def _pack_body(x_ref, o_ref):
    for c in range(PACK_VB // 1024):
        x = x_ref[:, c * 1024:(c + 1) * 1024]
        x8 = jnp.concatenate(
            [x[:, s * 128:(s + 1) * 128] for s in range(8)], axis=0)
        o_ref[0, c * 128:(c + 1) * 128, :] = jnp.transpose(x8)


_pack_call = pl.pallas_call(
    _pack_body,
    grid=(F_CAT, PACK_NVB),
    in_specs=[pl.BlockSpec((D, PACK_VB), lambda f, v: (f, v))],
    out_specs=pl.BlockSpec((1, PACK_PR, 128), lambda f, v: (f, v, 0)),
    out_shape=jax.ShapeDtypeStruct((F_CAT, PACK_NVB * PACK_PR, 128),
                                   jnp.float32),
)

NC = 2
NS = 16
NW = NC * NS
TOTAL_ROWS = F_CAT * B
R_PER_W = TOTAL_ROWS // NW
SUB = 128
NSUB = 13
CHUNK = SUB * NSUB
NCHUNK = R_PER_W // CHUNK
IDX_ROWS_PER_W = R_PER_W // SUB


def _sc_body(gide_hbm, gidl_hbm, emb_tab_hbm, lin_tab_hbm,
             emb_out_hbm, lin_out_hbm,
             idxe_v, idxl_v, rows_v, lin_v, sem_e, sem_l):
    wid = lax.axis_index("s") * NC + lax.axis_index("c")
    base = wid * R_PER_W
    idx_base = wid * IDX_ROWS_PER_W
    for c in range(NCHUNK):
        off = base + c * CHUNK
        pltpu.sync_copy(gide_hbm.at[pl.ds(idx_base + c * NSUB, NSUB)], idxe_v)
        pltpu.sync_copy(gidl_hbm.at[pl.ds(idx_base + c * NSUB, NSUB)], idxl_v)
        copies = []
        for j in range(NSUB):
            copies.append(pltpu.async_copy(
                emb_tab_hbm.at[idxe_v.at[j]],
                rows_v.at[pl.ds(j * SUB, SUB)], sem_e))
            copies.append(pltpu.async_copy(
                lin_tab_hbm.at[idxl_v.at[j]],
                lin_v.at[pl.ds(j * SUB, SUB)], sem_l))
        for cp in copies:
            cp.wait()
        pltpu.sync_copy(rows_v, emb_out_hbm.at[pl.ds(off, CHUNK)])
        pltpu.sync_copy(lin_v, lin_out_hbm.at[pl.ds(off, CHUNK)])


_sc_gather = functools.partial(
    pl.kernel,
    out_type=[
        jax.ShapeDtypeStruct((TOTAL_ROWS, D), jnp.float32),
        jax.ShapeDtypeStruct((TOTAL_ROWS,), jnp.float32),
    ],
    mesh=plsc.VectorSubcoreMesh(core_axis_name="c", subcore_axis_name="s"),
    scratch_types=[
        pltpu.VMEM((NSUB, SUB), jnp.int32),
        pltpu.VMEM((NSUB, SUB), jnp.int32),
        pltpu.VMEM((CHUNK, D), jnp.float32),
        pltpu.VMEM((CHUNK,), jnp.float32),
        pltpu.SemaphoreType.DMA,
        pltpu.SemaphoreType.DMA,
    ],
    compiler_params=pltpu.CompilerParams(use_tc_tiling_on_sc=False),
)(_sc_body)

BT = 2048
PRT = BT // 8


def _mlp_body(embp_ref, linp_ref, nump_ref, w1e_ref, w1n_ref, w2_ref, w3_ref,
              w4_ref, cwb_ref, b1_ref, b2_ref, b3_ref, b4_ref, out_ref):
    f32 = jnp.float32
    npk = nump_ref[...]
    acc = jnp.dot(npk, w1n_ref[...], preferred_element_type=f32)
    for f in range(F_CAT):
        acc = acc + jnp.dot(embp_ref[f], w1e_ref[f],
                            preferred_element_type=f32)
    h = jnp.maximum(acc + b1_ref[...], 0.0)
    h = jnp.maximum(jnp.dot(h, w2_ref[...], preferred_element_type=f32)
                    + b2_ref[...], 0.0)
    h = jnp.maximum(jnp.dot(h, w3_ref[...], preferred_element_type=f32)
                    + b3_ref[...], 0.0)
    z = jnp.dot(h, w4_ref[...], preferred_element_type=f32) + b4_ref[...]
    lin_s = linp_ref[0]
    for f in range(1, F_CAT):
        lin_s = lin_s + linp_ref[f]
    z = z + lin_s + jnp.dot(npk, cwb_ref[...], preferred_element_type=f32)
    out_ref[...] = 1.0 / (1.0 + jnp.exp(-z))


def _mlp_call(embp, linp, nump, w1e, w1n, w2, w3, w4, cwb, b1, b2, b3, b4):
    def full(shape):
        return pl.BlockSpec(shape, lambda *_: tuple(0 for _ in shape))

    return pl.pallas_call(
        _mlp_body,
        grid=(B // BT,),
        in_specs=[
            pl.BlockSpec((F_CAT, PRT, 128), lambda i: (0, i, 0)),
            pl.BlockSpec((F_CAT, PRT, 8), lambda i: (0, i, 0)),
            pl.BlockSpec((PRT, 128), lambda i: (i, 0)),
            full((F_CAT, 128, 512)),
            full((128, 512)),
            full((512, 256)),
            full((256, 128)),
            full((128, 8)),
            full((128, 8)),
            full((1, 512)),
            full((1, 256)),
            full((1, 128)),
            full((1, 8)),
        ],
        out_specs=pl.BlockSpec((PRT, 8), lambda i: (i, 0)),
        out_shape=jax.ShapeDtypeStruct((B // 8, 8), jnp.float32),
    )(embp, linp, nump, w1e, w1n, w2, w3, w4, cwb, b1, b2, b3, b4)


def kernel(cat, num, lin_tables, emb_tables, cont_w, W1, b1, W2, b2, W3, b3,
           W4, b4):
    emb_fm = jnp.transpose(emb_tables, (0, 2, 1)).reshape(F_CAT * D, V)
    packed = _pack_call(emb_fm)
    emb_tab = packed.reshape(EMB_ROWS_PAD, D)
    lin_flat = lin_tables.reshape(F_CAT * V)
    cat32 = cat.astype(jnp.int32)
    perm = cat32 - (cat32 & 1023) + ((cat32 & 127) << 3) + ((cat32 & 1023) >> 7)
    gidx_e = (perm + (jnp.arange(F_CAT, dtype=jnp.int32) * FSTRIDE)[:, None]
              ).reshape(TOTAL_ROWS // SUB, SUB)
    gidx_l = (cat32 + (jnp.arange(F_CAT, dtype=jnp.int32) * V)[:, None]
              ).reshape(TOTAL_ROWS // SUB, SUB)
    emb_rows, lin_rows = _sc_gather(gidx_e, gidx_l, emb_tab, lin_flat)
    f32 = jnp.float32
    eye8 = jnp.eye(8, dtype=f32)
    W1e3 = W1[:F_CAT * D].reshape(F_CAT, D, 64)
    Wb1e = jnp.einsum('jk,fdo->fjdko', eye8, W1e3).reshape(F_CAT, 128, 512)
    W1n = jnp.pad(W1[F_CAT * D:], ((0, 3), (0, 0)))
    Wb1n = jnp.einsum('jk,do->jdko', eye8, W1n).reshape(128, 512)
    Wb2 = jnp.einsum('jk,do->jdko', eye8, W2).reshape(512, 256)
    Wb3 = jnp.einsum('jk,do->jdko', eye8, W3).reshape(256, 128)
    Wb4 = jnp.einsum('jk,do->jdko', eye8, W4).reshape(128, 8)
    cwB = jnp.einsum('jk,c->jck', eye8, jnp.pad(cont_w, (0, 3))
                     ).reshape(128, 8)
    b1t = jnp.tile(b1, 8)[None]
    b2t = jnp.tile(b2, 8)[None]
    b3t = jnp.tile(b3, 8)[None]
    b4t = jnp.tile(b4, 8)[None]
    numP = jnp.pad(num.T, ((0, 0), (0, 3))).reshape(B // 8, 128)
    out8 = _mlp_call(
        emb_rows.reshape(F_CAT, B // 8, 128),
        lin_rows.reshape(F_CAT, B // 8, 8),
        numP, Wb1e, Wb1n, Wb2, Wb3, Wb4, cwB, b1t, b2t, b3t, b4t)
    return out8.reshape(B, 1)


# speedup vs baseline: 31.8521x; 1.0194x over previous
import functools

import jax
import jax.numpy as jnp
from jax import lax
from jax.experimental import pallas as pl
from jax.experimental.pallas import tpu as pltpu
from jax.experimental.pallas import tpu_sc as plsc

B = 16384
V = 100000
F_CAT = 26
F_NUM = 13
D = 16

PACK_VB = 33792
PACK_PR = PACK_VB // 8
PACK_NVB = 3
FSTRIDE = PACK_VB * PACK_NVB
PACK_ROWS = FSTRIDE // 8
EMB_ROWS_PAD = F_CAT * FSTRIDE


def _pack_body(x_ref, o_ref):
    for c in range(PACK_VB // 1024):
        x = x_ref[:, c * 1024:(c + 1) * 1024]
        x8 = jnp.concatenate(
            [x[:, s * 128:(s + 1) * 128] for s in range(8)], axis=0)
        o_ref[0, c * 128:(c + 1) * 128, :] = jnp.transpose(x8)


_pack_call = pl.pallas_call(
    _pack_body,
    grid=(F_CAT, PACK_NVB),
    in_specs=[pl.BlockSpec((D, PACK_VB), lambda f, v: (f, v))],
    out_specs=pl.BlockSpec((1, PACK_PR, 128), lambda f, v: (f, v, 0)),
    out_shape=jax.ShapeDtypeStruct((F_CAT, PACK_NVB * PACK_PR, 128),
                                   jnp.float32),
)

NC = 2
NS = 16
NW = NC * NS
TOTAL_ROWS = F_CAT * B
R_PER_W = TOTAL_ROWS // NW
SUB = 128
NSUB = 13
CHUNK = SUB * NSUB
NCHUNK = R_PER_W // CHUNK
IDX_ROWS_PER_W = R_PER_W // SUB


def _sc_body(gide_hbm, gidl_hbm, emb_tab_hbm, lin_tab_hbm,
             emb_out_hbm, lin_out_hbm,
             idxe_v, idxl_v, rows_v0, rows_v1, lin_v0, lin_v1,
             sem_e0, sem_e1, sem_l0, sem_l1):
    wid = lax.axis_index("s") * NC + lax.axis_index("c")
    base = wid * R_PER_W
    idx_base = wid * IDX_ROWS_PER_W
    pltpu.sync_copy(gide_hbm.at[pl.ds(idx_base, IDX_ROWS_PER_W)], idxe_v)
    pltpu.sync_copy(gidl_hbm.at[pl.ds(idx_base, IDX_ROWS_PER_W)], idxl_v)
    bufs = [(rows_v0, lin_v0, sem_e0, sem_l0),
            (rows_v1, lin_v1, sem_e1, sem_l1)]

    def fire(c):
        rv, lv, se, sl = bufs[c % 2]
        cps = []
        for j in range(NSUB):
            row = c * NSUB + j
            cps.append(pltpu.async_copy(
                emb_tab_hbm.at[idxe_v.at[row]],
                rv.at[pl.ds(j * SUB, SUB)], se))
            cps.append(pltpu.async_copy(
                lin_tab_hbm.at[idxl_v.at[row]],
                lv.at[pl.ds(j * SUB, SUB)], sl))
        return cps

    pend = fire(0)
    for c in range(NCHUNK):
        for cp in pend:
            cp.wait()
        if c + 1 < NCHUNK:
            pend = fire(c + 1)
        rv, lv = bufs[c % 2][0], bufs[c % 2][1]
        pltpu.sync_copy(rv, emb_out_hbm.at[pl.ds(base + c * CHUNK, CHUNK)])
        pltpu.sync_copy(lv, lin_out_hbm.at[pl.ds(base + c * CHUNK, CHUNK)])


_sc_gather = functools.partial(
    pl.kernel,
    out_type=[
        jax.ShapeDtypeStruct((TOTAL_ROWS, D), jnp.float32),
        jax.ShapeDtypeStruct((TOTAL_ROWS,), jnp.float32),
    ],
    mesh=plsc.VectorSubcoreMesh(core_axis_name="c", subcore_axis_name="s"),
    scratch_types=[
        pltpu.VMEM((IDX_ROWS_PER_W, SUB), jnp.int32),
        pltpu.VMEM((IDX_ROWS_PER_W, SUB), jnp.int32),
        pltpu.VMEM((CHUNK, D), jnp.float32),
        pltpu.VMEM((CHUNK, D), jnp.float32),
        pltpu.VMEM((CHUNK,), jnp.float32),
        pltpu.VMEM((CHUNK,), jnp.float32),
        pltpu.SemaphoreType.DMA,
        pltpu.SemaphoreType.DMA,
        pltpu.SemaphoreType.DMA,
        pltpu.SemaphoreType.DMA,
    ],
    compiler_params=pltpu.CompilerParams(use_tc_tiling_on_sc=False),
)(_sc_body)

BT = 2048
PRT = BT // 8


def _mlp_body(embp_ref, linp_ref, nump_ref, w1e_ref, w1n_ref, w2_ref, w3_ref,
              w4_ref, cwb_ref, b1_ref, b2_ref, b3_ref, b4_ref, out_ref):
    f32 = jnp.float32
    npk = nump_ref[...]
    acc = jnp.dot(npk, w1n_ref[...], preferred_element_type=f32)
    for f in range(F_CAT):
        acc = acc + jnp.dot(embp_ref[f], w1e_ref[f],
                            preferred_element_type=f32)
    h = jnp.maximum(acc + b1_ref[...], 0.0)
    h = jnp.maximum(jnp.dot(h, w2_ref[...], preferred_element_type=f32)
                    + b2_ref[...], 0.0)
    h = jnp.maximum(jnp.dot(h, w3_ref[...], preferred_element_type=f32)
                    + b3_ref[...], 0.0)
    z = jnp.dot(h, w4_ref[...], preferred_element_type=f32) + b4_ref[...]
    lin_s = linp_ref[0]
    for f in range(1, F_CAT):
        lin_s = lin_s + linp_ref[f]
    z = z + lin_s + jnp.dot(npk, cwb_ref[...], preferred_element_type=f32)
    out_ref[...] = 1.0 / (1.0 + jnp.exp(-z))


def _mlp_call(embp, linp, nump, w1e, w1n, w2, w3, w4, cwb, b1, b2, b3, b4):
    def full(shape):
        return pl.BlockSpec(shape, lambda *_: tuple(0 for _ in shape))

    return pl.pallas_call(
        _mlp_body,
        grid=(B // BT,),
        in_specs=[
            pl.BlockSpec((F_CAT, PRT, 128), lambda i: (0, i, 0)),
            pl.BlockSpec((F_CAT, PRT, 8), lambda i: (0, i, 0)),
            pl.BlockSpec((PRT, 128), lambda i: (i, 0)),
            full((F_CAT, 128, 512)),
            full((128, 512)),
            full((512, 256)),
            full((256, 128)),
            full((128, 8)),
            full((128, 8)),
            full((1, 512)),
            full((1, 256)),
            full((1, 128)),
            full((1, 8)),
        ],
        out_specs=pl.BlockSpec((PRT, 8), lambda i: (i, 0)),
        out_shape=jax.ShapeDtypeStruct((B // 8, 8), jnp.float32),
    )(embp, linp, nump, w1e, w1n, w2, w3, w4, cwb, b1, b2, b3, b4)


def kernel(cat, num, lin_tables, emb_tables, cont_w, W1, b1, W2, b2, W3, b3,
           W4, b4):
    emb_fm = jnp.transpose(emb_tables, (0, 2, 1)).reshape(F_CAT * D, V)
    packed = _pack_call(emb_fm)
    emb_tab = packed.reshape(EMB_ROWS_PAD, D)
    lin_flat = lin_tables.reshape(F_CAT * V)
    cat32 = cat.astype(jnp.int32)
    perm = cat32 - (cat32 & 1023) + ((cat32 & 127) << 3) + ((cat32 & 1023) >> 7)
    gidx_e = (perm + (jnp.arange(F_CAT, dtype=jnp.int32) * FSTRIDE)[:, None]
              ).reshape(TOTAL_ROWS // SUB, SUB)
    gidx_l = (cat32 + (jnp.arange(F_CAT, dtype=jnp.int32) * V)[:, None]
              ).reshape(TOTAL_ROWS // SUB, SUB)
    emb_rows, lin_rows = _sc_gather(gidx_e, gidx_l, emb_tab, lin_flat)
    f32 = jnp.float32
    eye8 = jnp.eye(8, dtype=f32)
    W1e3 = W1[:F_CAT * D].reshape(F_CAT, D, 64)
    Wb1e = jnp.einsum('jk,fdo->fjdko', eye8, W1e3).reshape(F_CAT, 128, 512)
    W1n = jnp.pad(W1[F_CAT * D:], ((0, 3), (0, 0)))
    Wb1n = jnp.einsum('jk,do->jdko', eye8, W1n).reshape(128, 512)
    Wb2 = jnp.einsum('jk,do->jdko', eye8, W2).reshape(512, 256)
    Wb3 = jnp.einsum('jk,do->jdko', eye8, W3).reshape(256, 128)
    Wb4 = jnp.einsum('jk,do->jdko', eye8, W4).reshape(128, 8)
    cwB = jnp.einsum('jk,c->jck', eye8, jnp.pad(cont_w, (0, 3))
                     ).reshape(128, 8)
    b1t = jnp.tile(b1, 8)[None]
    b2t = jnp.tile(b2, 8)[None]
    b3t = jnp.tile(b3, 8)[None]
    b4t = jnp.tile(b4, 8)[None]
    numP = jnp.pad(num.T, ((0, 0), (0, 3))).reshape(B // 8, 128)
    out8 = _mlp_call(
        emb_rows.reshape(F_CAT, B // 8, 128),
        lin_rows.reshape(F_CAT, B // 8, 8),
        numP, Wb1e, Wb1n, Wb2, Wb3, Wb4, cwB, b1t, b2t, b3t, b4t)
    return out8.reshape(B, 1)
